# baseline (speedup 1.0000x reference)
import functools

import jax
import jax.numpy as jnp
from jax.experimental import pallas as pl
from jax.experimental.pallas import tpu as pltpu


def _sum_body(x_ref, out_ref):
    i = pl.program_id(1)

    @pl.when(i == 0)
    def _init():
        out_ref[...] = jnp.zeros_like(out_ref)

    out_ref[0] += jnp.sum(x_ref[0], axis=0, keepdims=True)


def _route_body(s, sums_ref, key_ref, idx_ref, sim_ref, major_ref):
    eps = 1e-12
    xm = sums_ref[...] * (1.0 / s)
    ak = key_ref[...]
    b, na = xm.shape[0], ak.shape[0]
    akn = ak * jax.lax.rsqrt(jnp.maximum(jnp.sum(ak * ak, axis=1, keepdims=True), eps))
    xn = xm * jax.lax.rsqrt(jnp.maximum(jnp.sum(xm * xm, axis=1, keepdims=True), eps))
    sim = jnp.dot(xn, akn.T, preferred_element_type=jnp.float32)
    col = jax.lax.broadcasted_iota(jnp.int32, (b, na), 1)
    rowmax = jnp.max(sim, axis=1, keepdims=True)
    idx = jnp.min(jnp.where(sim == rowmax, col, na), axis=1, keepdims=True)
    counts = jnp.sum((idx == col).astype(jnp.int32), axis=0, keepdims=True)
    cmax = jnp.max(counts)
    major = jnp.min(jnp.where(counts == cmax, col[:1, :], na))
    idx_ref[...] = jnp.full_like(idx_ref, major)
    sim_ref[...] = jnp.full_like(
        sim_ref, jnp.sum(jnp.where(col[:1, :] == major, jnp.sum(sim, axis=0, keepdims=True), 0.0)) / b
    )
    major_ref[...] = jnp.full_like(major_ref, major)


def _mlp_body(mj_ref, x_ref, w1_ref, w2_ref, out_ref):
    x = x_ref[...]
    h = jnp.maximum(jnp.dot(x, w1_ref[0], preferred_element_type=jnp.float32), 0.0)
    a = jnp.maximum(jnp.dot(h, w2_ref[0], preferred_element_type=jnp.float32), 0.0)
    out_ref[...] = x + a


def _build(B, S, C, NA, H, interpret=False):
    chunk = 512
    nch = S // chunk

    sum_call = pl.pallas_call(
        _sum_body,
        grid=(B, nch),
        in_specs=[pl.BlockSpec((1, chunk, C), lambda b, i: (b, i, 0))],
        out_specs=pl.BlockSpec((1, 1, C), lambda b, i: (b, 0, 0)),
        out_shape=jax.ShapeDtypeStruct((B, 1, C), jnp.float32),
        interpret=interpret,
    )

    route_call = pl.pallas_call(
        functools.partial(_route_body, float(S)),
        out_shape=(
            jax.ShapeDtypeStruct((B, 1), jnp.int32),
            jax.ShapeDtypeStruct((1, 1), jnp.float32),
            jax.ShapeDtypeStruct((1, 1), jnp.int32),
        ),
        interpret=interpret,
    )

    rows = B * S
    mchunk = 512
    mlp_call = pl.pallas_call(
        _mlp_body,
        grid_spec=pltpu.PrefetchScalarGridSpec(
            num_scalar_prefetch=1,
            grid=(rows // mchunk,),
            in_specs=[
                pl.BlockSpec((mchunk, C), lambda i, mj: (i, 0)),
                pl.BlockSpec((1, C, H), lambda i, mj: (mj[0], 0, 0)),
                pl.BlockSpec((1, H, C), lambda i, mj: (mj[0], 0, 0)),
            ],
            out_specs=pl.BlockSpec((mchunk, C), lambda i, mj: (i, 0)),
        ),
        out_shape=jax.ShapeDtypeStruct((rows, C), jnp.float32),
        interpret=interpret,
    )
    return sum_call, route_call, mlp_call


def kernel(x_embed, adapter_key, W1, W2, interpret=False):
    B, S, C = x_embed.shape
    NA = adapter_key.shape[0]
    H = W1.shape[2]
    sum_call, route_call, mlp_call = _build(B, S, C, NA, H, interpret)
    sums = sum_call(x_embed).reshape(B, C)
    idx_full, rsim, major = route_call(sums, adapter_key)
    bias = mlp_call(major.reshape((1,)), x_embed.reshape(B * S, C), W1, W2)
    return idx_full, rsim.reshape(()), bias.reshape(B, S, C)
